# Initial kernel scaffold; baseline (speedup 1.0000x reference)
import jax
import jax.numpy as jnp
from jax.experimental import pallas as pl


def kernel(h, edge_index, edge_weight, W1, b1, gamma1, beta1, mean1, var1, a1, Wp, bp, W2, b2, gamma2, beta2, mean2, var2, a2):
    raise NotImplementedError("write your pallas kernel here")


# speedup vs baseline: 180.4584x; 180.4584x over previous
import math

import jax
import jax.numpy as jnp
from jax import lax
from jax.experimental import pallas as pl
from jax.experimental.pallas import tpu as pltpu

_EPS = 1e-5


def _adj_kernel(ei_ref, ew_ref, a1_ref, as_ref, bw_ref):
    C = a1_ref.shape[0]
    row = ei_ref[0:1, :]
    col = ei_ref[1:2, :]
    ew = ew_ref[0:1, :]
    ciota = lax.broadcasted_iota(jnp.int32, (C, 1), 0)
    occ = (col == ciota).astype(jnp.float32)
    orr = (row == ciota).astype(jnp.float32)
    ocw = occ * ew
    dn = (((1,), (1,)), ((), ()))
    bw = lax.dot_general(ocw, orr, dn, preferred_element_type=jnp.float32, precision=lax.Precision.HIGHEST)
    bs = lax.dot_general(occ, orr, dn, preferred_element_type=jnp.float32, precision=lax.Precision.HIGHEST)
    eye = (lax.broadcasted_iota(jnp.int32, (C, C), 0)
           == lax.broadcasted_iota(jnp.int32, (C, C), 1)).astype(jnp.float32)
    deg1 = jnp.sum(ocw, axis=1, keepdims=True) + 1.0
    degs = jnp.sum(occ, axis=1, keepdims=True) + 1.0
    dinv1 = jnp.where(deg1 > 0, lax.rsqrt(deg1), 0.0)
    dinvs = jnp.where(degs > 0, lax.rsqrt(degs), 0.0)
    dinv1_row = jnp.sum(dinv1 * eye, axis=0, keepdims=True)
    dinvs_row = jnp.sum(dinvs * eye, axis=0, keepdims=True)
    a1_ref[...] = dinv1 * (bw + eye) * dinv1_row
    as_ref[...] = dinvs * (bs + eye) * dinvs_row
    bw_ref[...] = bw


def _make_main_kernel(GB, C, F, K):
    kf = float(K)

    def _main_kernel(xb_ref, w1_ref, w2_ref, wp_ref, vecs_ref, a1_ref, as_ref,
                     bw_ref, out_ref, z_ref, xp_ref, m2_ref, p_ref):
        w1 = w1_ref[...]
        w2 = w2_ref[...]
        a1m = a1_ref[...]
        asm = as_ref[...]
        bwm = bw_ref[...]
        wp = wp_ref[...]
        b1v = vecs_ref[0:1, :]
        mn1 = vecs_ref[1:2, :]
        r1 = vecs_ref[2:3, :]
        g1 = vecs_ref[3:4, :]
        be1 = vecs_ref[4:5, :]
        b2v = vecs_ref[5:6, :]
        mn2 = vecs_ref[6:7, :]
        r2 = vecs_ref[7:8, :]
        g2 = vecs_ref[8:9, :]
        be2 = vecs_ref[9:10, :]
        a1s = vecs_ref[10:11, 0:1]
        a2s = vecs_ref[10:11, 1:2]
        bps = vecs_ref[10:11, 2:3]

        eye = (lax.broadcasted_iota(jnp.int32, (C, C), 0)
               == lax.broadcasted_iota(jnp.int32, (C, C), 1)).astype(jnp.float32)
        icf = lax.broadcasted_iota(jnp.int32, (C, 1), 0).astype(jnp.float32)
        irf = lax.broadcasted_iota(jnp.int32, (1, C), 1).astype(jnp.float32)
        zmask = (icf < kf).astype(jnp.float32)

        xw = jnp.dot(xb_ref[...].reshape(GB * C, F), w1,
                     preferred_element_type=jnp.float32)

        for g in range(GB):
            xwg = xw[g * C:(g + 1) * C, :]
            h1 = jnp.dot(a1m, xwg, preferred_element_type=jnp.float32, precision=lax.Precision.HIGHEST) + b1v
            xb = (h1 - mn1) * r1 * g1 + be1
            x1 = jnp.where(xb >= 0, xb, a1s * xb)
            s0 = jnp.dot(x1, wp, preferred_element_type=jnp.float32)
            spre = jnp.dot(asm, s0, preferred_element_type=jnp.float32, precision=lax.Precision.HIGHEST) + bps
            s = jnp.tanh(spre)
            sp_row = jnp.sum(spre * eye, axis=0, keepdims=True)
            cmp = (sp_row > spre) | ((sp_row == spre) & (irf < icf))
            rank = jnp.sum(cmp.astype(jnp.float32), axis=1, keepdims=True)
            mcol = (rank < kf).astype(jnp.float32)
            deg2 = mcol * (1.0 + jnp.dot(bwm, mcol,
                                         preferred_element_type=jnp.float32, precision=lax.Precision.HIGHEST))
            dinv2 = jnp.where(deg2 > 0, lax.rsqrt(deg2), 0.0)
            dinv2_row = jnp.sum(dinv2 * eye, axis=0, keepdims=True)
            m2_ref[g] = (bwm + eye) * dinv2 * dinv2_row
            rank_row = jnp.sum(rank * eye, axis=0, keepdims=True)
            p_ref[g] = (rank_row == icf).astype(jnp.float32)
            xp_ref[g * C:(g + 1) * C, :] = x1 * s

        y = jnp.dot(xp_ref[...], w2, preferred_element_type=jnp.float32)

        for g in range(GB):
            yg = y[g * C:(g + 1) * C, :]
            out2 = jnp.dot(m2_ref[g], yg, preferred_element_type=jnp.float32, precision=lax.Precision.HIGHEST) + b2v
            xb2 = (out2 - mn2) * r2 * g2 + be2
            act = jnp.where(xb2 >= 0, xb2, a2s * xb2)
            x2g = jnp.dot(p_ref[g], act, preferred_element_type=jnp.float32, precision=lax.Precision.HIGHEST)
            out_ref[g] = x2g
            z_ref[g:g + 1, :] = jnp.sum(x2g * zmask, axis=0, keepdims=True) / kf

    return _main_kernel


def kernel(h, edge_index, edge_weight, W1, b1, gamma1, beta1, mean1, var1, a1,
           Wp, bp, W2, b2, gamma2, beta2, mean2, var2, a2):
    Bb, C, F, Tt = h.shape
    G = Bb * Tt
    E = edge_index.shape[1]
    K = int(math.ceil(0.9 * C))
    NH = W1.shape[1]
    NO = W2.shape[1]
    GB = 32
    assert G % GB == 0

    X = jnp.transpose(h, (0, 3, 1, 2)).reshape(G, C, F)
    ew2d = edge_weight.reshape(1, E)

    A1, As, Bw = pl.pallas_call(
        _adj_kernel,
        out_shape=[jax.ShapeDtypeStruct((C, C), jnp.float32)] * 3,
    )(edge_index, ew2d)

    r1 = lax.rsqrt(var1 + _EPS)
    r2 = lax.rsqrt(var2 + _EPS)
    srow = jnp.zeros((NH,), jnp.float32)
    srow = srow.at[0].set(a1).at[1].set(a2).at[2].set(bp[0])
    zrow = jnp.zeros((NH,), jnp.float32)
    vecs = jnp.stack([b1, mean1, r1, gamma1, beta1, b2, mean2, r2, gamma2,
                      beta2, srow, zrow, zrow, zrow, zrow, zrow])

    grid = (G // GB,)
    x2p, z = pl.pallas_call(
        _make_main_kernel(GB, C, F, K),
        grid=grid,
        in_specs=[
            pl.BlockSpec((GB, C, F), lambda i: (i, 0, 0)),
            pl.BlockSpec((F, NH), lambda i: (0, 0)),
            pl.BlockSpec((NH, NO), lambda i: (0, 0)),
            pl.BlockSpec((NH, 1), lambda i: (0, 0)),
            pl.BlockSpec((16, NH), lambda i: (0, 0)),
            pl.BlockSpec((C, C), lambda i: (0, 0)),
            pl.BlockSpec((C, C), lambda i: (0, 0)),
            pl.BlockSpec((C, C), lambda i: (0, 0)),
        ],
        out_specs=[
            pl.BlockSpec((GB, C, NO), lambda i: (i, 0, 0)),
            pl.BlockSpec((GB, NO), lambda i: (i, 0)),
        ],
        out_shape=[
            jax.ShapeDtypeStruct((G, C, NO), jnp.float32),
            jax.ShapeDtypeStruct((G, NO), jnp.float32),
        ],
        scratch_shapes=[
            pltpu.VMEM((GB * C, NH), jnp.float32),
            pltpu.VMEM((GB, C, C), jnp.float32),
            pltpu.VMEM((GB, C, C), jnp.float32),
        ],
    )(X, W1, W2, Wp, vecs, A1, As, Bw)

    x2 = x2p[:, :K, :].reshape(G * K, NO)
    z_seq = jnp.transpose(z.reshape(Bb, Tt, NO), (0, 2, 1))
    return (x2, z_seq)


# speedup vs baseline: 251.4605x; 1.3935x over previous
import math

import jax
import jax.numpy as jnp
from jax import lax
from jax.experimental import pallas as pl
from jax.experimental.pallas import tpu as pltpu

_EPS = 1e-5
_HI = lax.Precision.HIGHEST
_MED = lax.Precision.HIGHEST


def _adj_kernel(ei_ref, ew_ref, a1_ref, as_ref, bw_ref):
    C = a1_ref.shape[0] // 2
    row = ei_ref[0:1, :]
    col = ei_ref[1:2, :]
    ew = ew_ref[0:1, :]
    ciota = lax.broadcasted_iota(jnp.int32, (C, 1), 0)
    occ = (col == ciota).astype(jnp.float32)
    orr = (row == ciota).astype(jnp.float32)
    ocw = occ * ew
    dn = (((1,), (1,)), ((), ()))
    bw = lax.dot_general(ocw, orr, dn, preferred_element_type=jnp.float32,
                         precision=_HI)
    bs = lax.dot_general(occ, orr, dn, preferred_element_type=jnp.float32,
                         precision=_HI)
    eye = (lax.broadcasted_iota(jnp.int32, (C, C), 0)
           == lax.broadcasted_iota(jnp.int32, (C, C), 1)).astype(jnp.float32)
    deg1 = jnp.sum(ocw, axis=1, keepdims=True) + 1.0
    degs = jnp.sum(occ, axis=1, keepdims=True) + 1.0
    dinv1 = jnp.where(deg1 > 0, lax.rsqrt(deg1), 0.0)
    dinvs = jnp.where(degs > 0, lax.rsqrt(degs), 0.0)
    dinv1_row = jnp.sum(dinv1 * eye, axis=0, keepdims=True)
    dinvs_row = jnp.sum(dinvs * eye, axis=0, keepdims=True)
    a1 = dinv1 * (bw + eye) * dinv1_row
    asx = dinvs * (bs + eye) * dinvs_row
    z = jnp.zeros((C, C), jnp.float32)
    for ref, m in ((a1_ref, a1), (as_ref, asx), (bw_ref, bw)):
        ref[0:C, 0:C] = m
        ref[0:C, C:2 * C] = z
        ref[C:2 * C, 0:C] = z
        ref[C:2 * C, C:2 * C] = m


def _make_main_kernel(GB, C, F, K):
    kf = float(K)
    C2 = 2 * C
    NP = GB // 2

    def _main_kernel(xb_ref, w1_ref, w2_ref, wp_ref, vecs_ref, a1_ref, as_ref,
                     bw_ref, out_ref, z_ref, xp_ref, rk_ref, di_ref):
        w1 = w1_ref[...]
        w2 = w2_ref[...]
        a1m = a1_ref[...]
        asm = as_ref[...]
        bwm = bw_ref[...]
        wp = wp_ref[...]
        b1v = vecs_ref[0:1, :]
        mn1 = vecs_ref[1:2, :]
        r1 = vecs_ref[2:3, :]
        g1 = vecs_ref[3:4, :]
        be1 = vecs_ref[4:5, :]
        b2v = vecs_ref[5:6, :]
        mn2 = vecs_ref[6:7, :]
        r2 = vecs_ref[7:8, :]
        g2 = vecs_ref[8:9, :]
        be2 = vecs_ref[9:10, :]
        a1s = vecs_ref[10:11, 0:1]
        a2s = vecs_ref[10:11, 1:2]
        bps = vecs_ref[10:11, 2:3]

        eye2 = (lax.broadcasted_iota(jnp.int32, (C2, C2), 0)
                == lax.broadcasted_iota(jnp.int32, (C2, C2), 1)).astype(jnp.float32)
        ic = lax.broadcasted_iota(jnp.int32, (C2, 1), 0)
        ir = lax.broadcasted_iota(jnp.int32, (1, C2), 1)
        icf = ic.astype(jnp.float32)
        irf = ir.astype(jnp.float32)
        same_blk = ((ic // C) == (ir // C))
        jloc = (ir - (ir // C) * C).astype(jnp.float32)
        zm = ((ic - (ic // C) * C) < K).astype(jnp.float32)
        bwi = bwm + eye2

        xw = jnp.dot(xb_ref[...].reshape(GB * C, F), w1,
                     preferred_element_type=jnp.float32)

        for p in range(NP):
            xwp = xw[p * C2:(p + 1) * C2, :]
            h1 = jnp.dot(a1m, xwp, preferred_element_type=jnp.float32,
                         precision=_HI) + b1v
            xb = (h1 - mn1) * r1 * g1 + be1
            x1 = jnp.where(xb >= 0, xb, a1s * xb)
            s0 = jnp.dot(x1, wp, preferred_element_type=jnp.float32)
            spre = jnp.dot(asm, s0, preferred_element_type=jnp.float32,
                           precision=_HI) + bps
            s = jnp.tanh(spre)
            sp_row = jnp.sum(spre * eye2, axis=0, keepdims=True)
            cmp = ((sp_row > spre) | ((sp_row == spre) & (irf < icf))) & same_blk
            rank = jnp.sum(cmp.astype(jnp.float32), axis=1, keepdims=True)
            mcol = (rank < kf).astype(jnp.float32)
            deg2 = mcol * (1.0 + jnp.dot(bwm, mcol,
                                         preferred_element_type=jnp.float32,
                                         precision=_HI))
            di_ref[p * C2:(p + 1) * C2, :] = jnp.where(deg2 > 0, lax.rsqrt(deg2), 0.0)
            rk_ref[p * C2:(p + 1) * C2, :] = rank
            xp_ref[p * C2:(p + 1) * C2, :] = x1 * s

        y = jnp.dot(xp_ref[...], w2, preferred_element_type=jnp.float32)

        for p in range(NP):
            dinv2 = di_ref[p * C2:(p + 1) * C2, :]
            rank = rk_ref[p * C2:(p + 1) * C2, :]
            yg = dinv2 * y[p * C2:(p + 1) * C2, :]
            out2 = dinv2 * jnp.dot(bwi, yg, preferred_element_type=jnp.float32,
                                   precision=_MED) + b2v
            xb2 = (out2 - mn2) * r2 * g2 + be2
            act = jnp.where(xb2 >= 0, xb2, a2s * xb2)
            pt = ((rank == jloc) & same_blk).astype(jnp.float32)
            x2g = lax.dot_general(pt, act, (((0,), (0,)), ((), ())),
                                  preferred_element_type=jnp.float32,
                                  precision=_MED)
            out_ref[2 * p] = x2g[0:C, :]
            out_ref[2 * p + 1] = x2g[C:C2, :]
            zs = x2g * zm
            z_ref[2 * p:2 * p + 1, :] = jnp.sum(zs[0:C, :], axis=0, keepdims=True) / kf
            z_ref[2 * p + 1:2 * p + 2, :] = jnp.sum(zs[C:C2, :], axis=0, keepdims=True) / kf

    return _main_kernel


def kernel(h, edge_index, edge_weight, W1, b1, gamma1, beta1, mean1, var1, a1,
           Wp, bp, W2, b2, gamma2, beta2, mean2, var2, a2):
    Bb, C, F, Tt = h.shape
    G = Bb * Tt
    E = edge_index.shape[1]
    K = int(math.ceil(0.9 * C))
    NH = W1.shape[1]
    NO = W2.shape[1]
    GB = 32
    assert G % GB == 0 and GB % 2 == 0

    X = jnp.transpose(h, (0, 3, 1, 2)).reshape(G, C, F)
    ew2d = edge_weight.reshape(1, E)

    A1b, Asb, Bwb = pl.pallas_call(
        _adj_kernel,
        out_shape=[jax.ShapeDtypeStruct((2 * C, 2 * C), jnp.float32)] * 3,
    )(edge_index, ew2d)

    r1 = lax.rsqrt(var1 + _EPS)
    r2 = lax.rsqrt(var2 + _EPS)
    srow = jnp.zeros((NH,), jnp.float32)
    srow = srow.at[0].set(a1).at[1].set(a2).at[2].set(bp[0])
    zrow = jnp.zeros((NH,), jnp.float32)
    vecs = jnp.stack([b1, mean1, r1, gamma1, beta1, b2, mean2, r2, gamma2,
                      beta2, srow, zrow, zrow, zrow, zrow, zrow])

    grid = (G // GB,)
    x2p, z = pl.pallas_call(
        _make_main_kernel(GB, C, F, K),
        grid=grid,
        in_specs=[
            pl.BlockSpec((GB, C, F), lambda i: (i, 0, 0)),
            pl.BlockSpec((F, NH), lambda i: (0, 0)),
            pl.BlockSpec((NH, NO), lambda i: (0, 0)),
            pl.BlockSpec((NH, 1), lambda i: (0, 0)),
            pl.BlockSpec((16, NH), lambda i: (0, 0)),
            pl.BlockSpec((2 * C, 2 * C), lambda i: (0, 0)),
            pl.BlockSpec((2 * C, 2 * C), lambda i: (0, 0)),
            pl.BlockSpec((2 * C, 2 * C), lambda i: (0, 0)),
        ],
        out_specs=[
            pl.BlockSpec((GB, C, NO), lambda i: (i, 0, 0)),
            pl.BlockSpec((GB, NO), lambda i: (i, 0)),
        ],
        out_shape=[
            jax.ShapeDtypeStruct((G, C, NO), jnp.float32),
            jax.ShapeDtypeStruct((G, NO), jnp.float32),
        ],
        scratch_shapes=[
            pltpu.VMEM((GB * C, NH), jnp.float32),
            pltpu.VMEM((GB * C, 1), jnp.float32),
            pltpu.VMEM((GB * C, 1), jnp.float32),
        ],
    )(X, W1, W2, Wp, vecs, A1b, Asb, Bwb)

    x2 = x2p[:, :K, :].reshape(G * K, NO)
    z_seq = jnp.transpose(z.reshape(Bb, Tt, NO), (0, 2, 1))
    return (x2, z_seq)
